# Initial kernel scaffold; baseline (speedup 1.0000x reference)
import jax
import jax.numpy as jnp
from jax.experimental import pallas as pl


def kernel(h1, h2, coord1, coord2, edge_index, coord_diff, edge_attr, W0, b0, W1, b1, W2):
    raise NotImplementedError("write your pallas kernel here")


# speedup vs baseline: 1.8830x; 1.8830x over previous
import functools

import jax
import jax.numpy as jnp
from jax import lax
from jax.experimental import pallas as pl
from jax.experimental.pallas import tpu as pltpu
from jax.experimental.pallas import tpu_sc as plsc

N = 10000
E = 160000
H = 256
NORM = 0.01

NPAD = 10240
CH = 128
NCHUNK = E // CH
NBLK = 1000
EBLK = 1000
NW_G = 32
NW_S = 16
NPW = NPAD // NW_S


def _precompute_body(h1_ref, h2_ref, w0a_ref, w0b_ref, a1_ref, a2_ref):
    a1_ref[...] = jnp.dot(h1_ref[...], w0a_ref[...],
                          preferred_element_type=jnp.float32)
    a2_ref[...] = jnp.dot(h2_ref[...], w0b_ref[...],
                          preferred_element_type=jnp.float32)


_precompute = pl.pallas_call(
    _precompute_body,
    grid=(N // NBLK,),
    in_specs=[
        pl.BlockSpec((NBLK, H), lambda i: (i, 0)),
        pl.BlockSpec((NBLK, H), lambda i: (i, 0)),
        pl.BlockSpec((H, H), lambda i: (0, 0)),
        pl.BlockSpec((H, H), lambda i: (0, 0)),
    ],
    out_specs=[
        pl.BlockSpec((NBLK, H), lambda i: (i, 0)),
        pl.BlockSpec((NBLK, H), lambda i: (i, 0)),
    ],
    out_shape=[
        jax.ShapeDtypeStruct((N, H), jnp.float32),
        jax.ShapeDtypeStruct((N, H), jnp.float32),
    ],
)


_MESH_G = plsc.VectorSubcoreMesh(core_axis_name="c", subcore_axis_name="s")


@functools.partial(
    pl.kernel,
    out_type=(jax.ShapeDtypeStruct((E, H), jnp.float32),
              jax.ShapeDtypeStruct((E, H), jnp.float32)),
    mesh=_MESH_G,
    scratch_types=[
        pltpu.VMEM((CH,), jnp.int32),
        pltpu.VMEM((CH,), jnp.int32),
        pltpu.VMEM((CH, H), jnp.float32),
        pltpu.VMEM((CH, H), jnp.float32),
        pltpu.SemaphoreType.DMA,
    ],
)
def _gather_rows(n1, n2, a1, a2, g1, g2, idx1_v, idx2_v, rows1_v, rows2_v,
                 sem):
    wid = lax.axis_index("s") * 2 + lax.axis_index("c")
    nchunks = NCHUNK // NW_G + jnp.where(wid < NCHUNK % NW_G, 1, 0)

    def body(i, carry):
        base = pl.multiple_of((wid + i * NW_G) * CH, CH)
        pltpu.sync_copy(n1.at[pl.ds(base, CH)], idx1_v)
        pltpu.sync_copy(n2.at[pl.ds(base, CH)], idx2_v)
        cp1 = pltpu.async_copy(a1.at[idx1_v], rows1_v, sem)
        cp2 = pltpu.async_copy(a2.at[idx2_v], rows2_v, sem)
        cp1.wait()
        cp2.wait()
        pltpu.sync_copy(rows1_v, g1.at[pl.ds(base, CH)])
        pltpu.sync_copy(rows2_v, g2.at[pl.ds(base, CH)])
        return carry

    lax.fori_loop(0, nchunks, body, 0)


def _mlp_body(g1, g2, ea, cd, w0c, b0, w1, b1, w2t, out):
    pre = g1[...] + g2[...] + ea[...] * w0c[...] + b0[...]
    x0 = pre * jax.nn.sigmoid(pre)
    pre1 = jnp.dot(x0, w1[...], preferred_element_type=jnp.float32) + b1[...]
    x1 = pre1 * jax.nn.sigmoid(pre1)
    m = jnp.sum(x1 * w2t[...], axis=1, keepdims=True)
    out[...] = cd[...] * m


_mlp = pl.pallas_call(
    _mlp_body,
    grid=(E // EBLK,),
    in_specs=[
        pl.BlockSpec((EBLK, H), lambda i: (i, 0)),
        pl.BlockSpec((EBLK, H), lambda i: (i, 0)),
        pl.BlockSpec((EBLK, 1), lambda i: (i, 0)),
        pl.BlockSpec((EBLK, 3), lambda i: (i, 0)),
        pl.BlockSpec((1, H), lambda i: (0, 0)),
        pl.BlockSpec((1, H), lambda i: (0, 0)),
        pl.BlockSpec((H, H), lambda i: (0, 0)),
        pl.BlockSpec((1, H), lambda i: (0, 0)),
        pl.BlockSpec((1, H), lambda i: (0, 0)),
    ],
    out_specs=pl.BlockSpec((EBLK, 3), lambda i: (i, 0)),
    out_shape=jax.ShapeDtypeStruct((E, 3), jnp.float32),
)


_MESH_S = plsc.VectorSubcoreMesh(core_axis_name="c", subcore_axis_name="s",
                                 num_cores=1)


@functools.partial(
    pl.kernel,
    out_type=tuple(jax.ShapeDtypeStruct((NPAD,), jnp.float32)
                   for _ in range(3)),
    mesh=_MESH_S,
    scratch_types=[
        pltpu.VMEM((CH,), jnp.int32),
        pltpu.VMEM((CH,), jnp.float32),
        pltpu.VMEM((CH,), jnp.float32),
        pltpu.VMEM((CH,), jnp.float32),
        pltpu.VMEM((NPW,), jnp.float32),
        pltpu.VMEM((NPW,), jnp.float32),
        pltpu.VMEM_SHARED((NPAD,), jnp.float32),
        pltpu.VMEM_SHARED((NPAD,), jnp.float32),
        pltpu.VMEM_SHARED((NPAD,), jnp.float32),
        pltpu.SemaphoreType.DMA,
    ],
)
def _scatter_combine(n1, tx, ty, tz, c1x, c1y, c1z, ox, oy, oz, idx_v,
                     tvx, tvy, tvz, node_v, c1_v, acc_x, acc_y, acc_z, sem):
    sid = lax.axis_index("s")
    accs = (acc_x, acc_y, acc_z)
    trans_c = (tx, ty, tz)
    tv_c = (tvx, tvy, tvz)
    c1_c = (c1x, c1y, c1z)
    out_c = (ox, oy, oz)

    def zbody(i, carry):
        node_v[pl.ds(pl.multiple_of(i * 16, 16), 16)] = jnp.zeros(
            (16,), jnp.float32)
        return carry

    lax.fori_loop(0, NPW // 16, zbody, 0)
    for comp in range(3):
        pltpu.sync_copy(node_v, accs[comp].at[pl.ds(sid * NPW, NPW)])
    plsc.subcore_barrier()

    nchunks = NCHUNK // NW_S + jnp.where(sid < NCHUNK % NW_S, 1, 0)

    def body(i, carry):
        base = pl.multiple_of((sid + i * NW_S) * CH, CH)
        pltpu.sync_copy(n1.at[pl.ds(base, CH)], idx_v)
        for comp in range(3):
            pltpu.sync_copy(trans_c[comp].at[pl.ds(base, CH)], tv_c[comp])
            pltpu.sync_copy(tv_c[comp], accs[comp].at[idx_v], add=True)
        return carry

    lax.fori_loop(0, nchunks, body, 0)
    plsc.subcore_barrier()

    for comp in range(3):
        pltpu.sync_copy(accs[comp].at[pl.ds(sid * NPW, NPW)], node_v)
        pltpu.sync_copy(c1_c[comp].at[pl.ds(sid * NPW, NPW)], c1_v)

        def fbody(i, carry):
            sl = pl.ds(pl.multiple_of(i * 16, 16), 16)
            node_v[sl] = c1_v[sl] + node_v[sl] * jnp.float32(NORM)
            return carry

        lax.fori_loop(0, NPW // 16, fbody, 0)
        pltpu.sync_copy(node_v, out_c[comp].at[pl.ds(sid * NPW, NPW)])


def kernel(h1, h2, coord1, coord2, edge_index, coord_diff, edge_attr, W0,
           b0, W1, b1, W2):
    del coord2
    n1 = edge_index[0].astype(jnp.int32)
    n2 = edge_index[1].astype(jnp.int32)
    w0a = W0[:H]
    w0b = W0[H:2 * H]
    w0c = W0[2 * H:].reshape(1, H)
    b0r = b0.reshape(1, H)
    b1r = b1.reshape(1, H)
    w2t = W2.reshape(1, H)

    a1, a2 = _precompute(h1, h2, w0a, w0b)
    g1, g2 = _gather_rows(n1, n2, a1, a2)
    trans = _mlp(g1, g2, edge_attr, coord_diff, w0c, b0r, W1, b1r, w2t)
    c1p = jnp.pad(coord1, ((0, NPAD - N), (0, 0)))
    ox, oy, oz = _scatter_combine(
        n1, trans[:, 0], trans[:, 1], trans[:, 2],
        c1p[:, 0], c1p[:, 1], c1p[:, 2])
    return jnp.stack([ox, oy, oz], axis=1)[:N]


# speedup vs baseline: 2.6489x; 1.4067x over previous
import functools

import jax
import jax.numpy as jnp
from jax import lax
from jax.experimental import pallas as pl
from jax.experimental.pallas import tpu as pltpu
from jax.experimental.pallas import tpu_sc as plsc

N = 10000
E = 160000
H = 256
NORM = 0.01

NPAD = 10240
CH = 128
NCHUNK = E // CH
NBLK = 1000
EBLK = 1280
NW_G = 32
NW_S = 16
NPW = NPAD // NW_S


def _precompute_body(h1_ref, h2_ref, w0a_ref, w0b_ref, a1_ref, a2_ref):
    a1_ref[...] = jnp.dot(h1_ref[...], w0a_ref[...],
                          preferred_element_type=jnp.float32)
    a2_ref[...] = jnp.dot(h2_ref[...], w0b_ref[...],
                          preferred_element_type=jnp.float32)


_precompute = pl.pallas_call(
    _precompute_body,
    grid=(N // NBLK,),
    in_specs=[
        pl.BlockSpec((NBLK, H), lambda i: (i, 0)),
        pl.BlockSpec((NBLK, H), lambda i: (i, 0)),
        pl.BlockSpec((H, H), lambda i: (0, 0)),
        pl.BlockSpec((H, H), lambda i: (0, 0)),
    ],
    out_specs=[
        pl.BlockSpec((NBLK, H), lambda i: (i, 0)),
        pl.BlockSpec((NBLK, H), lambda i: (i, 0)),
    ],
    out_shape=[
        jax.ShapeDtypeStruct((N, H), jnp.float32),
        jax.ShapeDtypeStruct((N, H), jnp.float32),
    ],
)


_MESH_G = plsc.VectorSubcoreMesh(core_axis_name="c", subcore_axis_name="s")


@functools.partial(
    pl.kernel,
    out_type=(jax.ShapeDtypeStruct((E, H), jnp.float32),
              jax.ShapeDtypeStruct((E, H), jnp.float32)),
    mesh=_MESH_G,
    scratch_types=[
        pltpu.VMEM((CH,), jnp.int32),
        pltpu.VMEM((CH,), jnp.int32),
        pltpu.VMEM((CH, H), jnp.float32),
        pltpu.VMEM((CH, H), jnp.float32),
        pltpu.SemaphoreType.DMA,
    ],
)
def _gather_rows(n1, n2, a1, a2, g1, g2, idx1_v, idx2_v, rows1_v, rows2_v,
                 sem):
    wid = lax.axis_index("s") * 2 + lax.axis_index("c")
    nchunks = NCHUNK // NW_G + jnp.where(wid < NCHUNK % NW_G, 1, 0)

    def body(i, carry):
        base = pl.multiple_of((wid + i * NW_G) * CH, CH)
        pltpu.sync_copy(n1.at[pl.ds(base, CH)], idx1_v)
        pltpu.sync_copy(n2.at[pl.ds(base, CH)], idx2_v)
        cp1 = pltpu.async_copy(a1.at[idx1_v], rows1_v, sem)
        cp2 = pltpu.async_copy(a2.at[idx2_v], rows2_v, sem)
        cp1.wait()
        cp2.wait()
        pltpu.sync_copy(rows1_v, g1.at[pl.ds(base, CH)])
        pltpu.sync_copy(rows2_v, g2.at[pl.ds(base, CH)])
        return carry

    lax.fori_loop(0, nchunks, body, 0)


def _mlp_body(g1, g2, ea, cd, w0c, b0, w1, b1, w2t, outx, outy, outz):
    pre = g1[...] + g2[...] + ea[...] * w0c[...] + b0[...]
    x0 = pre * jax.nn.sigmoid(pre)
    x0b = x0.astype(jnp.bfloat16)
    pre1 = jnp.dot(x0b, w1[...], preferred_element_type=jnp.float32) + b1[...]
    x1 = pre1 * jax.nn.sigmoid(pre1)
    m = jnp.sum(x1 * w2t[...], axis=1, keepdims=True)
    trans = cd[...] * m
    outx[...] = trans[:, 0].reshape(1, EBLK // CH, CH)
    outy[...] = trans[:, 1].reshape(1, EBLK // CH, CH)
    outz[...] = trans[:, 2].reshape(1, EBLK // CH, CH)


_mlp = pl.pallas_call(
    _mlp_body,
    grid=(E // EBLK,),
    in_specs=[
        pl.BlockSpec((EBLK, H), lambda i: (i, 0)),
        pl.BlockSpec((EBLK, H), lambda i: (i, 0)),
        pl.BlockSpec((EBLK, 1), lambda i: (i, 0)),
        pl.BlockSpec((EBLK, 3), lambda i: (i, 0)),
        pl.BlockSpec((1, H), lambda i: (0, 0)),
        pl.BlockSpec((1, H), lambda i: (0, 0)),
        pl.BlockSpec((H, H), lambda i: (0, 0)),
        pl.BlockSpec((1, H), lambda i: (0, 0)),
        pl.BlockSpec((1, H), lambda i: (0, 0)),
    ],
    out_specs=[pl.BlockSpec((1, EBLK // CH, CH), lambda i: (i, 0, 0))] * 3,
    out_shape=[jax.ShapeDtypeStruct((E // EBLK, EBLK // CH, CH),
                                    jnp.float32)] * 3,
)


_MESH_S = plsc.VectorSubcoreMesh(core_axis_name="c", subcore_axis_name="s",
                                 num_cores=1)


MAXC = 80
NCHUNKP = MAXC * NW_S
DRAIN = 8


@functools.partial(
    pl.kernel,
    out_type=tuple(jax.ShapeDtypeStruct((NPAD,), jnp.float32)
                   for _ in range(3)),
    mesh=_MESH_S,
    scratch_types=[
        pltpu.VMEM((MAXC, CH), jnp.int32),
        pltpu.VMEM((MAXC * CH,), jnp.float32),
        pltpu.VMEM((MAXC * CH,), jnp.float32),
        pltpu.VMEM((MAXC * CH,), jnp.float32),
        pltpu.VMEM((NPW,), jnp.float32),
        pltpu.VMEM((CH * 3,), jnp.float32),
        pltpu.VMEM_SHARED((NPAD,), jnp.float32),
        pltpu.VMEM_SHARED((NPAD,), jnp.float32),
        pltpu.VMEM_SHARED((NPAD,), jnp.float32),
        pltpu.SemaphoreType.DMA,
        pltpu.SemaphoreType.DMA,
    ],
)
def _scatter_combine(n1r, tx, ty, tz, c1x, c1y, c1z, ox, oy, oz, idx2d,
                     stx, sty, stz, zbuf, dummy_v, acc_x, acc_y,
                     acc_z, semL, semS):
    sid = lax.axis_index("s")
    accs = (acc_x, acc_y, acc_z)
    stg = (stx, sty, stz)
    t_c = (tx, ty, tz)
    c1_c = (c1x, c1y, c1z)
    out_c = (ox, oy, oz)

    start = pl.multiple_of(sid * MAXC, 8)
    nchunks = jnp.minimum(MAXC, NCHUNK - sid * MAXC)
    cps = [pltpu.async_copy(n1r.at[pl.ds(start, MAXC)], idx2d, semL)]
    for comp in range(3):
        cps.append(pltpu.async_copy(
            t_c[comp].at[pl.ds(start * CH, MAXC * CH)], stg[comp], semL))

    def zbody(i, carry):
        zbuf[pl.ds(pl.multiple_of(i * 16, 16), 16)] = jnp.zeros(
            (16,), jnp.float32)
        return carry

    lax.fori_loop(0, NPW // 16, zbody, 0)
    for comp in range(3):
        pltpu.sync_copy(zbuf, accs[comp].at[pl.ds(sid * NPW, NPW)])
    plsc.subcore_barrier()
    for cp in cps:
        cp.wait()

    def fire(j, carry):
        sbase = pl.multiple_of(j * CH, CH)
        for comp in range(3):
            pltpu.async_copy(stg[comp].at[pl.ds(sbase, CH)],
                             accs[comp].at[idx2d.at[j]], semS, add=True)

        @pl.when(j >= DRAIN)
        def _():
            pltpu.make_async_copy(tx.at[pl.ds(0, CH * 3)], dummy_v,
                                  semS).wait()

        return carry

    lax.fori_loop(0, nchunks, fire, 0)

    def drain(j, carry):
        pltpu.make_async_copy(tx.at[pl.ds(0, CH * 3)], dummy_v,
                              semS).wait()
        return carry

    lax.fori_loop(0, DRAIN, drain, 0)
    plsc.subcore_barrier()

    for comp in range(3):
        pltpu.sync_copy(accs[comp].at[pl.ds(sid * NPW, NPW)],
                        stg[comp].at[pl.ds(0, NPW)])
        pltpu.sync_copy(c1_c[comp].at[pl.ds(sid * NPW, NPW)], zbuf)

        def fbody(i, carry):
            sl = pl.ds(pl.multiple_of(i * 16, 16), 16)
            stg[comp][sl] = zbuf[sl] + stg[comp][sl] * jnp.float32(NORM)
            return carry

        lax.fori_loop(0, NPW // 16, fbody, 0)
        pltpu.sync_copy(stg[comp].at[pl.ds(0, NPW)],
                        out_c[comp].at[pl.ds(sid * NPW, NPW)])


def kernel(h1, h2, coord1, coord2, edge_index, coord_diff, edge_attr, W0,
           b0, W1, b1, W2):
    del coord2
    n1 = edge_index[0].astype(jnp.int32)
    n2 = edge_index[1].astype(jnp.int32)
    w0a = W0[:H]
    w0b = W0[H:2 * H]
    w0c = W0[2 * H:].reshape(1, H)
    b0r = b0.reshape(1, H)
    b1r = b1.reshape(1, H)
    w2t = W2.reshape(1, H)

    a1, a2 = _precompute(h1, h2, w0a, w0b)
    g1, g2 = _gather_rows(n1, n2, a1, a2)
    txp, typ, tzp = _mlp(g1, g2, edge_attr, coord_diff, w0c, b0r,
                         W1.astype(jnp.bfloat16), b1r, w2t)
    c1p = jnp.pad(coord1, ((0, NPAD - N), (0, 0)))
    n1r = jnp.pad(n1.reshape(NCHUNK, CH), ((0, NCHUNKP - NCHUNK), (0, 0)))
    epad = (NCHUNKP - NCHUNK) * CH
    ox, oy, oz = _scatter_combine(
        n1r, jnp.pad(txp.reshape(E), (0, epad)),
        jnp.pad(typ.reshape(E), (0, epad)),
        jnp.pad(tzp.reshape(E), (0, epad)),
        c1p[:, 0], c1p[:, 1], c1p[:, 2])
    return jnp.stack([ox, oy, oz], axis=1)[:N]
